# baseline (speedup 1.0000x reference)
import jax
import jax.numpy as jnp
from jax import lax
from jax.experimental import pallas as pl
from jax.experimental.pallas import tpu as pltpu
from jax.experimental.pallas import tpu_sc as plsc

NC = 2
NS = 16
NW = NC * NS

B, N, T, F = 4, 2048, 12, 64
A, K = 4, 32
ROW = T * F
NF = (1 + A) * F

ROWS_PER_W = (B * N) // NW
WPB = N // ROWS_PER_W
CHUNK = 8
NCHUNK = ROWS_PER_W // CHUNK
NSLOT = 3


def _body(src_hbm, table_hbm, pidx_hbm, out_hbm, idx_all,
          tables, pbuf2, rowbufs, semi0, semi1, semo):
    cid = lax.axis_index("c")
    sid = lax.axis_index("s")
    wid = sid * NC + cid
    b = wid // WPB
    n0b = (wid % WPB) * ROWS_PER_W

    pltpu.sync_copy(table_hbm.at[b], tables)
    for a in range(A):
        pltpu.sync_copy(pidx_hbm.at[pl.ds(a * N + n0b, ROWS_PER_W)],
                        idx_all.at[pl.ds(a * ROWS_PER_W, ROWS_PER_W)])

    semi = (semi0, semi1)

    def pass_desc(j, p):
        return pltpu.make_async_copy(
            src_hbm.at[pl.ds(b * N + n0b + j * CHUNK, CHUNK)],
            pbuf2.at[p], semi[p])

    def out_desc(j, i, slot):
        return pltpu.make_async_copy(
            rowbufs.at[slot], out_hbm.at[b, pl.ds(n0b + j * CHUNK + i, 1)],
            semo)

    def assemble_row(labs, p, i, slot):
        def one_t(t, _):
            base = pl.multiple_of(t * F, 16)
            for u in range(F // 16):
                rowbufs[slot, 0, t, pl.ds(u * 16, 16)] = \
                    pbuf2[p, i, pl.ds(base + u * 16, 16)]
            for a in range(A):
                for u in range(F // 16):
                    rowbufs[slot, 0, t, pl.ds((1 + a) * F + u * 16, 16)] = \
                        tables[labs[a], pl.ds(base + u * 16, 16)]
            return 0
        lax.fori_loop(0, T, one_t, 0)

    pass_desc(0, 0).start()

    def loop(jj, _):
        for p in (0, 1):
            j = 2 * jj + p
            jn = j + 1

            @pl.when(jn < NCHUNK)
            def _():
                pass_desc(jn, 1 - p).start()

            pass_desc(j, p).wait()

            labv = [idx_all[pl.ds(pl.multiple_of(a * ROWS_PER_W + jj * 16,
                                                 16), 16)]
                    for a in range(A)]
            for i in range(CHUNK):
                m = CHUNK * j + i
                slot = lax.rem(m, NSLOT)

                @pl.when(m >= NSLOT)
                def _():
                    out_desc(j, i, slot).wait()
                labs = [labv[a][p * CHUNK + i] for a in range(A)]
                assemble_row(labs, p, i, slot)
                out_desc(j, i, slot).start()
        return 0

    lax.fori_loop(0, NCHUNK // 2, loop, 0)
    for _ in range(NSLOT):
        out_desc(NCHUNK - 1, 0, 0).wait()


def kernel(output, subgraph_representation, node_labels):
    src2 = output.reshape(B * N, ROW)
    table = subgraph_representation.reshape(B, A * K, ROW)
    pidx = (node_labels.astype(jnp.int32)
            + jnp.arange(A, dtype=jnp.int32)[:, None] * K).reshape(A * N)
    mesh = plsc.VectorSubcoreMesh(
        core_axis_name="c", subcore_axis_name="s",
        num_cores=NC, num_subcores=NS,
    )
    return pl.kernel(
        _body,
        out_type=jax.ShapeDtypeStruct((B, N, T, NF), jnp.float32),
        mesh=mesh,
        scratch_types=[
            pltpu.VMEM((A * ROWS_PER_W,), jnp.int32),
            pltpu.VMEM((A * K, ROW), jnp.float32),
            pltpu.VMEM((2, CHUNK, ROW), jnp.float32),
            pltpu.VMEM((NSLOT, 1, T, NF), jnp.float32),
            pltpu.SemaphoreType.DMA,
            pltpu.SemaphoreType.DMA,
            pltpu.SemaphoreType.DMA,
        ],
    )(src2, table, pidx)


# speedup vs baseline: 1.0230x; 1.0230x over previous
import jax
import jax.numpy as jnp
from jax import lax
from jax.experimental import pallas as pl
from jax.experimental.pallas import tpu as pltpu
from jax.experimental.pallas import tpu_sc as plsc

NC = 2
NS = 16
NW = NC * NS

B, N, T, F = 4, 2048, 12, 64
A, K = 4, 32
ROW = T * F
NF = (1 + A) * F

ROWS_PER_W = (B * N) // NW
WPB = N // ROWS_PER_W
CHUNK = 8
NHALF = 2
HALF = CHUNK // NHALF
NCHUNK = ROWS_PER_W // CHUNK
NSLOT = 2


def _body(src_hbm, table_hbm, pidx_hbm, out_hbm, idx_all,
          pbuf2, gbufs2, rowbufs, semi0, semi1, semo):
    cid = lax.axis_index("c")
    sid = lax.axis_index("s")
    wid = sid * NC + cid
    b = wid // WPB
    n0b = (wid % WPB) * ROWS_PER_W

    for a in range(A):
        pltpu.sync_copy(pidx_hbm.at[pl.ds(a * N + n0b, ROWS_PER_W)],
                        idx_all.at[pl.ds(a * ROWS_PER_W, ROWS_PER_W)])

    semi = (semi0, semi1)

    def in_descs(j, p):
        descs = [pltpu.make_async_copy(
            src_hbm.at[pl.ds(b * N + n0b + j * CHUNK, CHUNK)],
            pbuf2.at[p], semi[p])]
        for a in range(A):
            descs.append(pltpu.make_async_copy(
                table_hbm.at[b].at[idx_all.at[pl.ds(a * ROWS_PER_W
                                                    + j * CHUNK, CHUNK)]],
                gbufs2.at[p, a], semi[p]))
        return descs

    def out_desc(j, h, slot):
        n0 = n0b + j * CHUNK + h * HALF
        return pltpu.make_async_copy(
            rowbufs.at[slot], out_hbm.at[b, pl.ds(n0, HALF)], semo)

    def assemble_half(p, h, slot):
        def one_row(i4, _):
            i = h * HALF + i4
            def one_t(t, _):
                base = pl.multiple_of(t * F, 16)
                for u in range(F // 16):
                    rowbufs[slot, i4, t, pl.ds(u * 16, 16)] = \
                        pbuf2[p, i, pl.ds(base + u * 16, 16)]
                for a in range(A):
                    for u in range(F // 16):
                        rowbufs[slot, i4, t,
                                pl.ds((1 + a) * F + u * 16, 16)] = \
                            gbufs2[p, a, i, pl.ds(base + u * 16, 16)]
                return 0
            return lax.fori_loop(0, T, one_t, 0)
        lax.fori_loop(0, HALF, one_row, 0)

    for d in in_descs(0, 0):
        d.start()

    def loop(jj, _):
        for p in (0, 1):
            j = 2 * jj + p
            jn = j + 1

            @pl.when(jn < NCHUNK)
            def _():
                for d in in_descs(jn, 1 - p):
                    d.start()

            for d in in_descs(j, p):
                d.wait()

            for h in range(NHALF):
                m = NHALF * j + h
                slot = lax.rem(m, NSLOT)

                @pl.when(m >= NSLOT)
                def _():
                    out_desc(j, h, slot).wait()
                assemble_half(p, h, slot)
                out_desc(j, h, slot).start()
        return 0

    lax.fori_loop(0, NCHUNK // 2, loop, 0)
    for _ in range(NSLOT):
        out_desc(NCHUNK - 1, NHALF - 1, 0).wait()


def kernel(output, subgraph_representation, node_labels):
    src2 = output.reshape(B * N, ROW)
    table = subgraph_representation.reshape(B, A * K, ROW)
    pidx = (node_labels.astype(jnp.int32)
            + jnp.arange(A, dtype=jnp.int32)[:, None] * K).reshape(A * N)
    mesh = plsc.VectorSubcoreMesh(
        core_axis_name="c", subcore_axis_name="s",
        num_cores=NC, num_subcores=NS,
    )
    return pl.kernel(
        _body,
        out_type=jax.ShapeDtypeStruct((B, N, T, NF), jnp.float32),
        mesh=mesh,
        scratch_types=[
            pltpu.VMEM((A * ROWS_PER_W,), jnp.int32),
            pltpu.VMEM((2, CHUNK, ROW), jnp.float32),
            pltpu.VMEM((2, A, CHUNK, ROW), jnp.float32),
            pltpu.VMEM((NSLOT, HALF, T, NF), jnp.float32),
            pltpu.SemaphoreType.DMA,
            pltpu.SemaphoreType.DMA,
            pltpu.SemaphoreType.DMA,
        ],
    )(src2, table, pidx)
